# speedup vs baseline: 1.6061x; 1.6061x over previous
import jax
import jax.numpy as jnp
from jax.experimental import pallas as pl
from jax.experimental.pallas import tpu as pltpu

BATCH = 16384
EMBED_DIM = 128
BLOCK_ROWS = 2048
GRID = BATCH // BLOCK_ROWS
N_TOTAL = float(BATCH * EMBED_DIM)


def _nll_kernel(cw_ref, xw_ref, out_ref, acc_ref):
    i = pl.program_id(0)

    @pl.when(i == 0)
    def _init():
        acc_ref[0] = 0.0
        acc_ref[1] = 0.0

    mul = cw_ref[...] * xw_ref[...]
    acc_ref[0] += jnp.sum(jnp.exp(mul))
    acc_ref[1] += jnp.sum(mul)

    @pl.when(i == GRID - 1)
    def _fini():
        out_ref[0] = N_TOTAL * jnp.log(acc_ref[0]) - acc_ref[1]


@jax.jit
def kernel(center_word, context_word, center_emb, context_emb):
    del center_emb, context_emb
    out = pl.pallas_call(
        _nll_kernel,
        grid=(GRID,),
        in_specs=[
            pl.BlockSpec((BLOCK_ROWS, EMBED_DIM), lambda i: (i, 0)),
            pl.BlockSpec((BLOCK_ROWS, EMBED_DIM), lambda i: (i, 0)),
        ],
        out_specs=pl.BlockSpec(memory_space=pltpu.SMEM),
        out_shape=jax.ShapeDtypeStruct((1,), jnp.float32),
        scratch_shapes=[pltpu.SMEM((2,), jnp.float32)],
    )(center_word, context_word)
    return out[0]
